# Initial kernel scaffold; baseline (speedup 1.0000x reference)
import jax
import jax.numpy as jnp
from jax.experimental import pallas as pl


def kernel(x, edge_index, batch, W1l, W1r, b1, W2l, W2r, b2, Wfc, bfc):
    raise NotImplementedError("write your pallas kernel here")


# speedup vs baseline: 3.8653x; 3.8653x over previous
import functools

import jax
import jax.numpy as jnp
from jax import lax
from jax.experimental import pallas as pl
from jax.experimental.pallas import tpu as pltpu
from jax.experimental.pallas import tpu_sc as plsc

N_NODES = 10000
N_PAD = 10240
E_EDGES = 320000
CH = 128
NCH = 80
NCHP = 88
EPT = E_EDGES // 32
D = 128
D_OUT = 64
N_GRAPHS = 64
ROWS_PER_TILE = N_PAD // 16


def _mesh():
  return plsc.VectorSubcoreMesh(core_axis_name="c", subcore_axis_name="s",
                                num_cores=2, num_subcores=16)


SCH = 8
NSCH = NCH // SCH


@functools.cache
def _make_sc_agg(with_cnt: bool):
  out_type = [jax.ShapeDtypeStruct((2, N_PAD, D), jnp.float32)]
  if with_cnt:
    out_type.append(jax.ShapeDtypeStruct((2, N_PAD), jnp.float32))
  scratch = [
      pltpu.VMEM((2 * SCH, CH), jnp.int32),
      pltpu.VMEM((CH, D), jnp.float32),
      pltpu.VMEM_SHARED((N_PAD, D), jnp.float32),
      pltpu.SemaphoreType.DMA,
  ]
  if with_cnt:
    scratch += [
        pltpu.VMEM((N_PAD,), jnp.float32),
        pltpu.VMEM_SHARED((16, N_PAD), jnp.float32),
        pltpu.VMEM((ROWS_PER_TILE,), jnp.float32),
        pltpu.VMEM((ROWS_PER_TILE,), jnp.float32),
    ]

  def body(x_hbm, ei_hbm, zrow_hbm, *refs):
    if with_cnt:
      (zhist_hbm, out_hbm, cnt_hbm, idxv, rows, acc, sem, hist, hs, tmp,
       ssum) = refs
    else:
      (out_hbm, idxv, rows, acc, sem) = refs
    c = lax.axis_index("c")
    s = lax.axis_index("s")
    wid = c * 16 + s

    pltpu.sync_copy(zrow_hbm, acc.at[pl.ds(s * ROWS_PER_TILE, ROWS_PER_TILE)])
    if with_cnt:
      pltpu.sync_copy(zhist_hbm, hist)
    plsc.subcore_barrier()

    ones16 = jnp.ones((16,), jnp.float32)

    def step(j, carry):
      pltpu.sync_copy(ei_hbm.at[wid, pl.ds(j * 2 * SCH, 2 * SCH)], idxv)
      for k in range(SCH):
        cp = pltpu.async_copy(x_hbm.at[idxv.at[2 * k]], rows, sem)
        if with_cnt:
          for m in range(CH // 16):
            d = idxv[2 * k + 1, pl.ds(m * 16, 16)]
            plsc.addupdate_scatter(hist, [d], ones16)
        cp.wait()
        pltpu.sync_copy(rows, acc.at[idxv.at[2 * k + 1]], add=True)
      return carry

    lax.fori_loop(0, NSCH, step, 0)
    if with_cnt:
      pltpu.sync_copy(hist, hs.at[s])
    plsc.subcore_barrier()

    sl = pl.ds(s * ROWS_PER_TILE, ROWS_PER_TILE)
    pltpu.sync_copy(acc.at[sl], out_hbm.at[c, sl])

    if with_cnt:
      nv = ROWS_PER_TILE // 16

      def zero(v, carry2):
        vs = pl.ds(v * 16, 16)
        ssum[vs] = jnp.zeros((16,), jnp.float32)
        return carry2

      def red(t, carry2):
        pltpu.sync_copy(hs.at[t, sl], tmp)

        def add(v, carry3):
          vs = pl.ds(v * 16, 16)
          ssum[vs] = ssum[vs] + tmp[vs]
          return carry3

        lax.fori_loop(0, nv, add, 0)
        return carry2

      lax.fori_loop(0, nv, zero, 0)
      lax.fori_loop(0, 16, red, 0)
      pltpu.sync_copy(ssum, cnt_hbm.at[c, sl])

  return pl.kernel(body, out_type=out_type, mesh=_mesh(),
                   scratch_types=scratch,
                   compiler_params=pltpu.CompilerParams(
                       needs_layout_passes=False))


BLK = 1024
GRID = N_PAD // BLK


def _tc_layer1_body(p_ref, c_ref, x_ref, wl_ref, wr_ref, b_ref, h_ref):
  agg = p_ref[0] + p_ref[1]
  cnt = c_ref[0] + c_ref[1]
  mean = agg / jnp.maximum(cnt, 1.0)
  h = (jnp.dot(mean, wl_ref[...], preferred_element_type=jnp.float32)
       + jnp.dot(x_ref[...], wr_ref[...], preferred_element_type=jnp.float32)
       + b_ref[...])
  h_ref[...] = jnp.maximum(h, 0.0)


_tc_layer1 = pl.pallas_call(
    _tc_layer1_body,
    grid=(GRID,),
    in_specs=[
        pl.BlockSpec((2, BLK, D), lambda i: (0, i, 0)),
        pl.BlockSpec((2, BLK, 1), lambda i: (0, i, 0)),
        pl.BlockSpec((BLK, D), lambda i: (i, 0)),
        pl.BlockSpec((D, D), lambda i: (0, 0)),
        pl.BlockSpec((D, D), lambda i: (0, 0)),
        pl.BlockSpec((1, D), lambda i: (0, 0)),
    ],
    out_specs=pl.BlockSpec((BLK, D), lambda i: (i, 0)),
    out_shape=jax.ShapeDtypeStruct((N_PAD, D), jnp.float32),
    compiler_params=pltpu.CompilerParams(
        dimension_semantics=("arbitrary",)),
)


def _tc_layer2_body(p_ref, c_ref, h_ref, b3_ref, wl_ref, wr_ref, b_ref,
                    wfc_ref, bfc_ref, out_ref, ps_ref, gc_ref):
  i = pl.program_id(0)

  @pl.when(i == 0)
  def _init():
    ps_ref[...] = jnp.zeros_like(ps_ref)
    gc_ref[...] = jnp.zeros_like(gc_ref)

  agg = p_ref[0] + p_ref[1]
  cnt = c_ref[0] + c_ref[1]
  mean = agg / jnp.maximum(cnt, 1.0)
  h2 = (jnp.dot(mean, wl_ref[...], preferred_element_type=jnp.float32)
        + jnp.dot(h_ref[...], wr_ref[...], preferred_element_type=jnp.float32)
        + b_ref[...])
  b = b3_ref[0, 0, :]
  onehot = (lax.broadcasted_iota(jnp.int32, (N_GRAPHS, BLK), 0)
            == b[None, :]).astype(jnp.float32)
  ps_ref[...] += jnp.dot(onehot, h2, preferred_element_type=jnp.float32)
  gc_ref[...] += jnp.broadcast_to(
      jnp.sum(onehot, axis=1, keepdims=True), (N_GRAPHS, D))

  @pl.when(i == GRID - 1)
  def _fin():
    pooled = ps_ref[...] / jnp.maximum(gc_ref[...], 1.0)
    out_ref[...] = (jnp.dot(pooled, wfc_ref[...],
                            preferred_element_type=jnp.float32)
                    + bfc_ref[...])


_tc_layer2 = pl.pallas_call(
    _tc_layer2_body,
    grid=(GRID,),
    in_specs=[
        pl.BlockSpec((2, BLK, D), lambda i: (0, i, 0)),
        pl.BlockSpec((2, BLK, 1), lambda i: (0, i, 0)),
        pl.BlockSpec((BLK, D), lambda i: (i, 0)),
        pl.BlockSpec((1, 1, BLK), lambda i: (i, 0, 0)),
        pl.BlockSpec((D, D), lambda i: (0, 0)),
        pl.BlockSpec((D, D), lambda i: (0, 0)),
        pl.BlockSpec((1, D), lambda i: (0, 0)),
        pl.BlockSpec((D, D_OUT), lambda i: (0, 0)),
        pl.BlockSpec((1, D_OUT), lambda i: (0, 0)),
    ],
    out_specs=pl.BlockSpec((N_GRAPHS, D_OUT), lambda i: (0, 0)),
    out_shape=jax.ShapeDtypeStruct((N_GRAPHS, D_OUT), jnp.float32),
    scratch_shapes=[
        pltpu.VMEM((N_GRAPHS, D), jnp.float32),
        pltpu.VMEM((N_GRAPHS, D), jnp.float32),
    ],
    compiler_params=pltpu.CompilerParams(
        dimension_semantics=("arbitrary",)),
)


@jax.jit
def kernel(x, edge_index, batch, W1l, W1r, b1, W2l, W2r, b2, Wfc, bfc):
  src = edge_index[0]
  dst = edge_index[1]
  tpad = NCHP * CH - EPT
  srcp = jnp.concatenate(
      [src.reshape(32, EPT), jnp.zeros((32, tpad), jnp.int32)],
      axis=1).reshape(32, NCHP, CH)
  dstp = jnp.concatenate(
      [dst.reshape(32, EPT), jnp.full((32, tpad), N_NODES, jnp.int32)],
      axis=1).reshape(32, NCHP, CH)
  ei = jnp.stack([srcp, dstp], axis=2).reshape(32, 2 * NCHP, CH)
  xpad = jnp.concatenate(
      [x, jnp.zeros((N_PAD - N_NODES, D), jnp.float32)])
  batchp = jnp.concatenate(
      [batch, jnp.full((N_PAD - N_NODES,), N_NODES, jnp.int32)]
  ).reshape(GRID, 1, BLK)
  zrow = jnp.zeros((ROWS_PER_TILE, D), jnp.float32)
  zhist = jnp.zeros((N_PAD,), jnp.float32)

  part1, cnt = _make_sc_agg(True)(x, ei, zrow, zhist)
  cntp = cnt.reshape(2, N_PAD, 1)
  h = _tc_layer1(part1, cntp, xpad, W1l, W1r, b1.reshape(1, D))
  (part2,) = _make_sc_agg(False)(h, ei, zrow)
  out = _tc_layer2(part2, cntp, h, batchp, W2l, W2r, b2.reshape(1, D),
                   Wfc, bfc.reshape(1, D_OUT))
  return out


# speedup vs baseline: 4.0192x; 1.0398x over previous
import functools

import jax
import jax.numpy as jnp
from jax import lax
from jax.experimental import pallas as pl
from jax.experimental.pallas import tpu as pltpu
from jax.experimental.pallas import tpu_sc as plsc

N_NODES = 10000
N_PAD = 10240
E_EDGES = 320000
CH = 128
NCH = 80
NCHP = 88
EPT = E_EDGES // 32
D = 128
D_OUT = 64
N_GRAPHS = 64
ROWS_PER_TILE = N_PAD // 16


def _mesh():
  return plsc.VectorSubcoreMesh(core_axis_name="c", subcore_axis_name="s",
                                num_cores=2, num_subcores=16)


SCH = 8
NSCH = NCH // SCH


@functools.cache
def _make_sc_agg(with_cnt: bool):
  out_type = [jax.ShapeDtypeStruct((2, N_PAD, D), jnp.float32)]
  if with_cnt:
    out_type.append(jax.ShapeDtypeStruct((2, N_PAD), jnp.float32))
  scratch = [
      pltpu.VMEM((2 * SCH, CH), jnp.int32),
      pltpu.VMEM((CH, D) if with_cnt else (2, CH, D), jnp.float32),
      pltpu.VMEM_SHARED((N_PAD, D), jnp.float32),
      pltpu.SemaphoreType.DMA,
  ]
  if with_cnt:
    scratch += [
        pltpu.VMEM((N_PAD,), jnp.float32),
        pltpu.VMEM_SHARED((16, N_PAD), jnp.float32),
        pltpu.VMEM((ROWS_PER_TILE,), jnp.float32),
        pltpu.VMEM((ROWS_PER_TILE,), jnp.float32),
    ]
  else:
    scratch.append(pltpu.SemaphoreType.DMA)

  def body(x_hbm, ei_hbm, zrow_hbm, *refs):
    if with_cnt:
      (zhist_hbm, out_hbm, cnt_hbm, idxv, rows, acc, sem, hist, hs, tmp,
       ssum) = refs
    else:
      (out_hbm, idxv, rows, acc, sem, sem1) = refs
    c = lax.axis_index("c")
    s = lax.axis_index("s")
    wid = c * 16 + s

    pltpu.sync_copy(zrow_hbm, acc.at[pl.ds(s * ROWS_PER_TILE, ROWS_PER_TILE)])
    if with_cnt:
      pltpu.sync_copy(zhist_hbm, hist)
    plsc.subcore_barrier()

    ones16 = jnp.ones((16,), jnp.float32)

    def step(j, carry):
      pltpu.sync_copy(ei_hbm.at[wid, pl.ds(j * 2 * SCH, 2 * SCH)], idxv)
      if with_cnt:
        for k in range(SCH):
          cp = pltpu.async_copy(x_hbm.at[idxv.at[2 * k]], rows, sem)
          for m in range(CH // 16):
            d = idxv[2 * k + 1, pl.ds(m * 16, 16)]
            plsc.addupdate_scatter(hist, [d], ones16)
          cp.wait()
          pltpu.sync_copy(rows, acc.at[idxv.at[2 * k + 1]], add=True)
      else:
        sems = (sem, sem1)
        pltpu.async_copy(x_hbm.at[idxv.at[0]], rows.at[0], sems[0])
        for k in range(SCH):
          b = k % 2
          pltpu.make_async_copy(x_hbm.at[pl.ds(0, CH)], rows.at[b],
                                sems[b]).wait()
          if k < SCH - 1:
            pltpu.async_copy(x_hbm.at[idxv.at[2 * k + 2]], rows.at[1 - b],
                             sems[1 - b])
          pltpu.sync_copy(rows.at[b], acc.at[idxv.at[2 * k + 1]], add=True)
      return carry

    lax.fori_loop(0, NSCH, step, 0)
    if with_cnt:
      pltpu.sync_copy(hist, hs.at[s])
    plsc.subcore_barrier()

    sl = pl.ds(s * ROWS_PER_TILE, ROWS_PER_TILE)
    pltpu.sync_copy(acc.at[sl], out_hbm.at[c, sl])

    if with_cnt:
      nv = ROWS_PER_TILE // 16

      def zero(v, carry2):
        vs = pl.ds(v * 16, 16)
        ssum[vs] = jnp.zeros((16,), jnp.float32)
        return carry2

      def red(t, carry2):
        pltpu.sync_copy(hs.at[t, sl], tmp)

        def add(v, carry3):
          vs = pl.ds(v * 16, 16)
          ssum[vs] = ssum[vs] + tmp[vs]
          return carry3

        lax.fori_loop(0, nv, add, 0)
        return carry2

      lax.fori_loop(0, nv, zero, 0)
      lax.fori_loop(0, 16, red, 0)
      pltpu.sync_copy(ssum, cnt_hbm.at[c, sl])

  return pl.kernel(body, out_type=out_type, mesh=_mesh(),
                   scratch_types=scratch,
                   compiler_params=pltpu.CompilerParams(
                       needs_layout_passes=False))


BLK = 1024
GRID = N_PAD // BLK


def _tc_layer1_body(p_ref, c_ref, x_ref, wl_ref, wr_ref, b_ref, h_ref):
  agg = p_ref[0] + p_ref[1]
  cnt = c_ref[0] + c_ref[1]
  mean = agg / jnp.maximum(cnt, 1.0)
  h = (jnp.dot(mean, wl_ref[...], preferred_element_type=jnp.float32)
       + jnp.dot(x_ref[...], wr_ref[...], preferred_element_type=jnp.float32)
       + b_ref[...])
  h_ref[...] = jnp.maximum(h, 0.0)


_tc_layer1 = pl.pallas_call(
    _tc_layer1_body,
    grid=(GRID,),
    in_specs=[
        pl.BlockSpec((2, BLK, D), lambda i: (0, i, 0)),
        pl.BlockSpec((2, BLK, 1), lambda i: (0, i, 0)),
        pl.BlockSpec((BLK, D), lambda i: (i, 0)),
        pl.BlockSpec((D, D), lambda i: (0, 0)),
        pl.BlockSpec((D, D), lambda i: (0, 0)),
        pl.BlockSpec((1, D), lambda i: (0, 0)),
    ],
    out_specs=pl.BlockSpec((BLK, D), lambda i: (i, 0)),
    out_shape=jax.ShapeDtypeStruct((N_PAD, D), jnp.float32),
    compiler_params=pltpu.CompilerParams(
        dimension_semantics=("arbitrary",)),
)


def _tc_layer2_body(p_ref, c_ref, h_ref, b3_ref, wl_ref, wr_ref, b_ref,
                    wfc_ref, bfc_ref, out_ref, ps_ref, gc_ref):
  i = pl.program_id(0)

  @pl.when(i == 0)
  def _init():
    ps_ref[...] = jnp.zeros_like(ps_ref)
    gc_ref[...] = jnp.zeros_like(gc_ref)

  agg = p_ref[0] + p_ref[1]
  cnt = c_ref[0] + c_ref[1]
  mean = agg / jnp.maximum(cnt, 1.0)
  h2 = (jnp.dot(mean, wl_ref[...], preferred_element_type=jnp.float32)
        + jnp.dot(h_ref[...], wr_ref[...], preferred_element_type=jnp.float32)
        + b_ref[...])
  b = b3_ref[0, 0, :]
  onehot = (lax.broadcasted_iota(jnp.int32, (N_GRAPHS, BLK), 0)
            == b[None, :]).astype(jnp.float32)
  ps_ref[...] += jnp.dot(onehot, h2, preferred_element_type=jnp.float32)
  gc_ref[...] += jnp.broadcast_to(
      jnp.sum(onehot, axis=1, keepdims=True), (N_GRAPHS, D))

  @pl.when(i == GRID - 1)
  def _fin():
    pooled = ps_ref[...] / jnp.maximum(gc_ref[...], 1.0)
    out_ref[...] = (jnp.dot(pooled, wfc_ref[...],
                            preferred_element_type=jnp.float32)
                    + bfc_ref[...])


_tc_layer2 = pl.pallas_call(
    _tc_layer2_body,
    grid=(GRID,),
    in_specs=[
        pl.BlockSpec((2, BLK, D), lambda i: (0, i, 0)),
        pl.BlockSpec((2, BLK, 1), lambda i: (0, i, 0)),
        pl.BlockSpec((BLK, D), lambda i: (i, 0)),
        pl.BlockSpec((1, 1, BLK), lambda i: (i, 0, 0)),
        pl.BlockSpec((D, D), lambda i: (0, 0)),
        pl.BlockSpec((D, D), lambda i: (0, 0)),
        pl.BlockSpec((1, D), lambda i: (0, 0)),
        pl.BlockSpec((D, D_OUT), lambda i: (0, 0)),
        pl.BlockSpec((1, D_OUT), lambda i: (0, 0)),
    ],
    out_specs=pl.BlockSpec((N_GRAPHS, D_OUT), lambda i: (0, 0)),
    out_shape=jax.ShapeDtypeStruct((N_GRAPHS, D_OUT), jnp.float32),
    scratch_shapes=[
        pltpu.VMEM((N_GRAPHS, D), jnp.float32),
        pltpu.VMEM((N_GRAPHS, D), jnp.float32),
    ],
    compiler_params=pltpu.CompilerParams(
        dimension_semantics=("arbitrary",)),
)


@jax.jit
def kernel(x, edge_index, batch, W1l, W1r, b1, W2l, W2r, b2, Wfc, bfc):
  src = edge_index[0]
  dst = edge_index[1]
  tpad = NCHP * CH - EPT
  srcp = jnp.concatenate(
      [src.reshape(32, EPT), jnp.zeros((32, tpad), jnp.int32)],
      axis=1).reshape(32, NCHP, CH)
  dstp = jnp.concatenate(
      [dst.reshape(32, EPT), jnp.full((32, tpad), N_NODES, jnp.int32)],
      axis=1).reshape(32, NCHP, CH)
  ei = jnp.stack([srcp, dstp], axis=2).reshape(32, 2 * NCHP, CH)
  xpad = jnp.concatenate(
      [x, jnp.zeros((N_PAD - N_NODES, D), jnp.float32)])
  batchp = jnp.concatenate(
      [batch, jnp.full((N_PAD - N_NODES,), N_NODES, jnp.int32)]
  ).reshape(GRID, 1, BLK)
  zrow = jnp.zeros((ROWS_PER_TILE, D), jnp.float32)
  zhist = jnp.zeros((N_PAD,), jnp.float32)

  part1, cnt = _make_sc_agg(True)(x, ei, zrow, zhist)
  cntp = cnt.reshape(2, N_PAD, 1)
  h = _tc_layer1(part1, cntp, xpad, W1l, W1r, b1.reshape(1, D))
  (part2,) = _make_sc_agg(False)(h, ei, zrow)
  out = _tc_layer2(part2, cntp, h, batchp, W2l, W2r, b2.reshape(1, D),
                   Wfc, bfc.reshape(1, D_OUT))
  return out


# speedup vs baseline: 4.2057x; 1.0464x over previous
import functools

import jax
import jax.numpy as jnp
from jax import lax
from jax.experimental import pallas as pl
from jax.experimental.pallas import tpu as pltpu
from jax.experimental.pallas import tpu_sc as plsc

N_NODES = 10000
N_PAD = 10240
E_EDGES = 320000
CH = 128
NCH = 80
NCHP = 88
EPT = E_EDGES // 32
D = 128
D_OUT = 64
N_GRAPHS = 64
ROWS_PER_TILE = N_PAD // 16


def _mesh():
  return plsc.VectorSubcoreMesh(core_axis_name="c", subcore_axis_name="s",
                                num_cores=2, num_subcores=16)


SCH = 8
NSCH = NCH // SCH


@functools.cache
def _make_sc_agg(with_cnt: bool):
  out_type = [jax.ShapeDtypeStruct((2, N_PAD, D), jnp.float32)]
  if with_cnt:
    out_type.append(jax.ShapeDtypeStruct((32, N_PAD), jnp.float32))
  scratch = [
      pltpu.VMEM((2 * SCH, CH), jnp.int32),
      pltpu.VMEM((2, CH, D), jnp.float32),
      pltpu.VMEM_SHARED((N_PAD, D), jnp.float32),
      pltpu.SemaphoreType.DMA,
      pltpu.SemaphoreType.DMA,
  ]
  if with_cnt:
    scratch.append(pltpu.VMEM((N_PAD,), jnp.float32))

  def body(x_hbm, ei_hbm, zrow_hbm, *refs):
    if with_cnt:
      (zhist_hbm, out_hbm, cnt_hbm, idxv, rows, acc, sem, sem1,
       hist) = refs
    else:
      (out_hbm, idxv, rows, acc, sem, sem1) = refs
    c = lax.axis_index("c")
    s = lax.axis_index("s")
    wid = c * 16 + s

    pltpu.sync_copy(zrow_hbm, acc.at[pl.ds(s * ROWS_PER_TILE, ROWS_PER_TILE)])
    if with_cnt:
      pltpu.sync_copy(zhist_hbm, hist)
    plsc.subcore_barrier()

    ones16 = jnp.ones((16,), jnp.float32)

    def step(j, carry):
      pltpu.sync_copy(ei_hbm.at[wid, pl.ds(j * 2 * SCH, 2 * SCH)], idxv)
      sems = (sem, sem1)
      pltpu.async_copy(x_hbm.at[idxv.at[0]], rows.at[0], sems[0])
      for k in range(SCH):
        b = k % 2
        pltpu.make_async_copy(x_hbm.at[pl.ds(0, CH)], rows.at[b],
                              sems[b]).wait()
        if k < SCH - 1:
          pltpu.async_copy(x_hbm.at[idxv.at[2 * k + 2]], rows.at[1 - b],
                           sems[1 - b])
        if with_cnt:
          for m in range(CH // 16):
            d = idxv[2 * k + 1, pl.ds(m * 16, 16)]
            plsc.addupdate_scatter(hist, [d], ones16)
        pltpu.sync_copy(rows.at[b], acc.at[idxv.at[2 * k + 1]], add=True)
      return carry

    lax.fori_loop(0, NSCH, step, 0)
    if with_cnt:
      pltpu.sync_copy(hist, cnt_hbm.at[wid])
    plsc.subcore_barrier()

    sl = pl.ds(s * ROWS_PER_TILE, ROWS_PER_TILE)
    pltpu.sync_copy(acc.at[sl], out_hbm.at[c, sl])

  return pl.kernel(body, out_type=out_type, mesh=_mesh(),
                   scratch_types=scratch,
                   compiler_params=pltpu.CompilerParams(
                       needs_layout_passes=False))


BLK = 1024
GRID = N_PAD // BLK


def _tc_layer1_body(p_ref, c_ref, x_ref, wl_ref, wr_ref, b_ref, h_ref):
  agg = p_ref[0] + p_ref[1]
  cnt = jnp.dot(c_ref[...], jnp.ones((32, 1), jnp.float32),
                preferred_element_type=jnp.float32)
  mean = agg / jnp.maximum(cnt, 1.0)
  h = (jnp.dot(mean, wl_ref[...], preferred_element_type=jnp.float32)
       + jnp.dot(x_ref[...], wr_ref[...], preferred_element_type=jnp.float32)
       + b_ref[...])
  h_ref[...] = jnp.maximum(h, 0.0)


_tc_layer1 = pl.pallas_call(
    _tc_layer1_body,
    grid=(GRID,),
    in_specs=[
        pl.BlockSpec((2, BLK, D), lambda i: (0, i, 0)),
        pl.BlockSpec((BLK, 32), lambda i: (i, 0)),
        pl.BlockSpec((BLK, D), lambda i: (i, 0)),
        pl.BlockSpec((D, D), lambda i: (0, 0)),
        pl.BlockSpec((D, D), lambda i: (0, 0)),
        pl.BlockSpec((1, D), lambda i: (0, 0)),
    ],
    out_specs=pl.BlockSpec((BLK, D), lambda i: (i, 0)),
    out_shape=jax.ShapeDtypeStruct((N_PAD, D), jnp.float32),
    compiler_params=pltpu.CompilerParams(
        dimension_semantics=("arbitrary",)),
)


def _tc_layer2_body(p_ref, c_ref, h_ref, b3_ref, wl_ref, wr_ref, b_ref,
                    wfc_ref, bfc_ref, out_ref, ps_ref, gc_ref):
  i = pl.program_id(0)

  @pl.when(i == 0)
  def _init():
    ps_ref[...] = jnp.zeros_like(ps_ref)
    gc_ref[...] = jnp.zeros_like(gc_ref)

  agg = p_ref[0] + p_ref[1]
  cnt = jnp.dot(c_ref[...], jnp.ones((32, 1), jnp.float32),
                preferred_element_type=jnp.float32)
  mean = agg / jnp.maximum(cnt, 1.0)
  h2 = (jnp.dot(mean, wl_ref[...], preferred_element_type=jnp.float32)
        + jnp.dot(h_ref[...], wr_ref[...], preferred_element_type=jnp.float32)
        + b_ref[...])
  b = b3_ref[0, 0, :]
  onehot = (lax.broadcasted_iota(jnp.int32, (N_GRAPHS, BLK), 0)
            == b[None, :]).astype(jnp.float32)
  ps_ref[...] += jnp.dot(onehot, h2, preferred_element_type=jnp.float32)
  gc_ref[...] += jnp.broadcast_to(
      jnp.sum(onehot, axis=1, keepdims=True), (N_GRAPHS, D))

  @pl.when(i == GRID - 1)
  def _fin():
    pooled = ps_ref[...] / jnp.maximum(gc_ref[...], 1.0)
    out_ref[...] = (jnp.dot(pooled, wfc_ref[...],
                            preferred_element_type=jnp.float32)
                    + bfc_ref[...])


_tc_layer2 = pl.pallas_call(
    _tc_layer2_body,
    grid=(GRID,),
    in_specs=[
        pl.BlockSpec((2, BLK, D), lambda i: (0, i, 0)),
        pl.BlockSpec((BLK, 32), lambda i: (i, 0)),
        pl.BlockSpec((BLK, D), lambda i: (i, 0)),
        pl.BlockSpec((1, 1, BLK), lambda i: (i, 0, 0)),
        pl.BlockSpec((D, D), lambda i: (0, 0)),
        pl.BlockSpec((D, D), lambda i: (0, 0)),
        pl.BlockSpec((1, D), lambda i: (0, 0)),
        pl.BlockSpec((D, D_OUT), lambda i: (0, 0)),
        pl.BlockSpec((1, D_OUT), lambda i: (0, 0)),
    ],
    out_specs=pl.BlockSpec((N_GRAPHS, D_OUT), lambda i: (0, 0)),
    out_shape=jax.ShapeDtypeStruct((N_GRAPHS, D_OUT), jnp.float32),
    scratch_shapes=[
        pltpu.VMEM((N_GRAPHS, D), jnp.float32),
        pltpu.VMEM((N_GRAPHS, D), jnp.float32),
    ],
    compiler_params=pltpu.CompilerParams(
        dimension_semantics=("arbitrary",)),
)


@jax.jit
def kernel(x, edge_index, batch, W1l, W1r, b1, W2l, W2r, b2, Wfc, bfc):
  src = edge_index[0]
  dst = edge_index[1]
  tpad = NCHP * CH - EPT
  srcp = jnp.concatenate(
      [src.reshape(32, EPT), jnp.zeros((32, tpad), jnp.int32)],
      axis=1).reshape(32, NCHP, CH)
  dstp = jnp.concatenate(
      [dst.reshape(32, EPT), jnp.full((32, tpad), N_NODES, jnp.int32)],
      axis=1).reshape(32, NCHP, CH)
  ei = jnp.stack([srcp, dstp], axis=2).reshape(32, 2 * NCHP, CH)
  xpad = jnp.concatenate(
      [x, jnp.zeros((N_PAD - N_NODES, D), jnp.float32)])
  batchp = jnp.concatenate(
      [batch, jnp.full((N_PAD - N_NODES,), N_NODES, jnp.int32)]
  ).reshape(GRID, 1, BLK)
  zrow = jnp.zeros((ROWS_PER_TILE, D), jnp.float32)
  zhist = jnp.zeros((N_PAD,), jnp.float32)

  part1, cnt = _make_sc_agg(True)(x, ei, zrow, zhist)
  cntp = cnt.T
  h = _tc_layer1(part1, cntp, xpad, W1l, W1r, b1.reshape(1, D))
  (part2,) = _make_sc_agg(False)(h, ei, zrow)
  out = _tc_layer2(part2, cntp, h, batchp, W2l, W2r, b2.reshape(1, D),
                   Wfc, bfc.reshape(1, D_OUT))
  return out
